# baseline (speedup 1.0000x reference)
import functools

import jax
import jax.numpy as jnp
from jax import lax
from jax.experimental import pallas as pl
from jax.experimental.pallas import tpu as pltpu
from jax.experimental.pallas import tpu_sc as plsc

_EMBED_DIM = 64
_HIDDEN = 128
_TAGS = 128

_GATHER_WINDOW = 128
_TBL_BLK = 20480


def _sc_gather(table, xt3):
    nblk, _, w = xt3.shape
    n = nblk * w
    d = table.shape[1]
    mesh = plsc.VectorSubcoreMesh(core_axis_name="core", subcore_axis_name="subcore")

    @functools.partial(
        pl.kernel,
        out_type=jax.ShapeDtypeStruct((n, d), table.dtype),
        mesh=mesh,
    )
    def gather_kernel(tbl_hbm, idx_hbm, out_hbm):
        def body(idx_vmem, out_vmem):
            pltpu.sync_copy(tbl_hbm.at[idx_vmem.at[0, 0]], out_vmem)

        pltpu.emit_pipeline(
            body,
            grid=(nblk,),
            in_specs=[pl.BlockSpec((1, 1, w), lambda i: (i, 0, 0))],
            out_specs=[pl.BlockSpec((w, d), lambda i: (i, 0))],
            core_axis_name=("core", "subcore"),
            dimension_semantics=(pltpu.PARALLEL,),
        )(idx_hbm, out_hbm)

    return gather_kernel(table, xt3)


def _mlp_body(et_ref, w1_ref, b1_ref, w2_ref, b2_ref, w3_ref, b3_ref, o_ref):
    h = jax.lax.dot_general(
        et_ref[...].astype(jnp.bfloat16), w1_ref[...].astype(jnp.bfloat16),
        (((0,), (0,)), ((), ())),
        preferred_element_type=jnp.float32,
    )
    h = jnp.maximum(h + b1_ref[...], 0.0)
    h = jnp.dot(h.astype(jnp.bfloat16), w2_ref[...].astype(jnp.bfloat16),
                preferred_element_type=jnp.float32)
    h = jnp.maximum(h + b2_ref[...], 0.0)
    o_ref[...] = jnp.dot(h.astype(jnp.bfloat16), w3_ref[...].astype(jnp.bfloat16),
                         preferred_element_type=jnp.float32) + b3_ref[...]


def _tc_table_mlp(embt, W1, b1, W2, b2, W3, b3):
    v = embt.shape[1]
    grid = pl.cdiv(v, _TBL_BLK)
    return pl.pallas_call(
        _mlp_body,
        grid=(grid,),
        in_specs=[
            pl.BlockSpec((_EMBED_DIM, _TBL_BLK), lambda i: (0, i)),
            pl.BlockSpec((_EMBED_DIM, _HIDDEN), lambda i: (0, 0)),
            pl.BlockSpec((1, _HIDDEN), lambda i: (0, 0)),
            pl.BlockSpec((_HIDDEN, _HIDDEN), lambda i: (0, 0)),
            pl.BlockSpec((1, _HIDDEN), lambda i: (0, 0)),
            pl.BlockSpec((_HIDDEN, _TAGS), lambda i: (0, 0)),
            pl.BlockSpec((1, _TAGS), lambda i: (0, 0)),
        ],
        out_specs=pl.BlockSpec((_TBL_BLK, _TAGS), lambda i: (i, 0)),
        out_shape=jax.ShapeDtypeStruct((v, _TAGS), jnp.float32),
    )(embt, W1, b1.reshape(1, -1), W2, b2.reshape(1, -1), W3, b3.reshape(1, -1))


def kernel(x, emb, W1, b1, W2, b2, W3, b3):
    b, l = x.shape
    w = _GATHER_WINDOW
    xt3 = x.astype(jnp.int32).T.reshape(l * b // w, 1, w)
    table = _tc_table_mlp(emb.T, W1, b1, W2, b2, W3, b3)
    out = _sc_gather(table, xt3)
    return out.reshape(l, b, _TAGS).transpose(1, 0, 2)


# speedup vs baseline: 1.1260x; 1.1260x over previous
import functools

import jax
import jax.numpy as jnp
from jax import lax
from jax.experimental import pallas as pl
from jax.experimental.pallas import tpu as pltpu
from jax.experimental.pallas import tpu_sc as plsc

_EMBED_DIM = 64
_HIDDEN = 128
_TAGS = 128

_GATHER_WINDOW = 400
_TBL_BLK = 20480


def _sc_gather(table, xt3):
    nblk, _, w = xt3.shape
    n = nblk * w
    d = table.shape[1]
    mesh = plsc.VectorSubcoreMesh(core_axis_name="core", subcore_axis_name="subcore")

    @functools.partial(
        pl.kernel,
        out_type=jax.ShapeDtypeStruct((n, d), table.dtype),
        mesh=mesh,
    )
    def gather_kernel(tbl_hbm, idx_hbm, out_hbm):
        def body(idx_vmem, out_vmem):
            pltpu.sync_copy(tbl_hbm.at[idx_vmem.at[0, 0]], out_vmem)

        pltpu.emit_pipeline(
            body,
            grid=(nblk,),
            in_specs=[pl.BlockSpec((1, 1, w), lambda i: (i, 0, 0))],
            out_specs=[pl.BlockSpec((w, d), lambda i: (i, 0))],
            core_axis_name=("core", "subcore"),
            dimension_semantics=(pltpu.PARALLEL,),
        )(idx_hbm, out_hbm)

    return gather_kernel(table, xt3)


def _mlp_body(et_ref, w1_ref, b1_ref, w2_ref, b2_ref, w3_ref, b3_ref, o_ref):
    h = jax.lax.dot_general(
        et_ref[...].astype(jnp.bfloat16), w1_ref[...].astype(jnp.bfloat16),
        (((0,), (0,)), ((), ())),
        preferred_element_type=jnp.float32,
    )
    h = jnp.maximum(h + b1_ref[...], 0.0)
    h = jnp.dot(h.astype(jnp.bfloat16), w2_ref[...].astype(jnp.bfloat16),
                preferred_element_type=jnp.float32)
    h = jnp.maximum(h + b2_ref[...], 0.0)
    o_ref[...] = jnp.dot(h.astype(jnp.bfloat16), w3_ref[...].astype(jnp.bfloat16),
                         preferred_element_type=jnp.float32) + b3_ref[...]


def _tc_table_mlp(embt, W1, b1, W2, b2, W3, b3):
    v = embt.shape[1]
    grid = pl.cdiv(v, _TBL_BLK)
    return pl.pallas_call(
        _mlp_body,
        grid=(grid,),
        in_specs=[
            pl.BlockSpec((_EMBED_DIM, _TBL_BLK), lambda i: (0, i)),
            pl.BlockSpec((_EMBED_DIM, _HIDDEN), lambda i: (0, 0)),
            pl.BlockSpec((1, _HIDDEN), lambda i: (0, 0)),
            pl.BlockSpec((_HIDDEN, _HIDDEN), lambda i: (0, 0)),
            pl.BlockSpec((1, _HIDDEN), lambda i: (0, 0)),
            pl.BlockSpec((_HIDDEN, _TAGS), lambda i: (0, 0)),
            pl.BlockSpec((1, _TAGS), lambda i: (0, 0)),
        ],
        out_specs=pl.BlockSpec((_TBL_BLK, _TAGS), lambda i: (i, 0)),
        out_shape=jax.ShapeDtypeStruct((v, _TAGS), jnp.float32),
    )(embt, W1, b1.reshape(1, -1), W2, b2.reshape(1, -1), W3, b3.reshape(1, -1))


def kernel(x, emb, W1, b1, W2, b2, W3, b3):
    b, l = x.shape
    w = _GATHER_WINDOW
    xt3 = x.astype(jnp.int32).T.reshape(l * b // w, 1, w)
    table = _tc_table_mlp(emb.T, W1, b1, W2, b2, W3, b3)
    out = _sc_gather(table, xt3)
    return out.reshape(l, b, _TAGS).transpose(1, 0, 2)


# speedup vs baseline: 1.1322x; 1.0055x over previous
import functools

import jax
import jax.numpy as jnp
from jax import lax
from jax.experimental import pallas as pl
from jax.experimental.pallas import tpu as pltpu
from jax.experimental.pallas import tpu_sc as plsc

_EMBED_DIM = 64
_HIDDEN = 128
_TAGS = 128

_GATHER_WINDOW = 256
_TBL_BLK = 20480


def _sc_gather(table, xt3):
    nblk, _, w = xt3.shape
    n = nblk * w
    d = table.shape[1]
    mesh = plsc.VectorSubcoreMesh(core_axis_name="core", subcore_axis_name="subcore")

    @functools.partial(
        pl.kernel,
        out_type=jax.ShapeDtypeStruct((n, d), table.dtype),
        mesh=mesh,
    )
    def gather_kernel(tbl_hbm, idx_hbm, out_hbm):
        def body(idx_vmem, out_vmem):
            pltpu.sync_copy(tbl_hbm.at[idx_vmem.at[0, 0]], out_vmem)

        pltpu.emit_pipeline(
            body,
            grid=(nblk,),
            in_specs=[pl.BlockSpec((1, 1, w), lambda i: (i, 0, 0))],
            out_specs=[pl.BlockSpec((w, d), lambda i: (i, 0))],
            core_axis_name=("core", "subcore"),
            dimension_semantics=(pltpu.PARALLEL,),
        )(idx_hbm, out_hbm)

    return gather_kernel(table, xt3)


def _mlp_body(et_ref, w1_ref, b1_ref, w2_ref, b2_ref, w3_ref, b3_ref, o_ref):
    h = jax.lax.dot_general(
        et_ref[...].astype(jnp.bfloat16), w1_ref[...].astype(jnp.bfloat16),
        (((0,), (0,)), ((), ())),
        preferred_element_type=jnp.float32,
    )
    h = jnp.maximum(h + b1_ref[...], 0.0)
    h = jnp.dot(h.astype(jnp.bfloat16), w2_ref[...].astype(jnp.bfloat16),
                preferred_element_type=jnp.float32)
    h = jnp.maximum(h + b2_ref[...], 0.0)
    o_ref[...] = jnp.dot(h.astype(jnp.bfloat16), w3_ref[...].astype(jnp.bfloat16),
                         preferred_element_type=jnp.float32) + b3_ref[...]


def _tc_table_mlp(embt, W1, b1, W2, b2, W3, b3):
    v = embt.shape[1]
    grid = pl.cdiv(v, _TBL_BLK)
    return pl.pallas_call(
        _mlp_body,
        grid=(grid,),
        in_specs=[
            pl.BlockSpec((_EMBED_DIM, _TBL_BLK), lambda i: (0, i)),
            pl.BlockSpec((_EMBED_DIM, _HIDDEN), lambda i: (0, 0)),
            pl.BlockSpec((1, _HIDDEN), lambda i: (0, 0)),
            pl.BlockSpec((_HIDDEN, _HIDDEN), lambda i: (0, 0)),
            pl.BlockSpec((1, _HIDDEN), lambda i: (0, 0)),
            pl.BlockSpec((_HIDDEN, _TAGS), lambda i: (0, 0)),
            pl.BlockSpec((1, _TAGS), lambda i: (0, 0)),
        ],
        out_specs=pl.BlockSpec((_TBL_BLK, _TAGS), lambda i: (i, 0)),
        out_shape=jax.ShapeDtypeStruct((v, _TAGS), jnp.float32),
    )(embt, W1, b1.reshape(1, -1), W2, b2.reshape(1, -1), W3, b3.reshape(1, -1))


def kernel(x, emb, W1, b1, W2, b2, W3, b3):
    b, l = x.shape
    w = _GATHER_WINDOW
    xt3 = x.astype(jnp.int32).T.reshape(l * b // w, 1, w)
    table = _tc_table_mlp(emb.T, W1, b1, W2, b2, W3, b3)
    out = _sc_gather(table, xt3)
    return out.reshape(l, b, _TAGS).transpose(1, 0, 2)
